# baseline (device time: 76855 ns/iter reference)
import jax
import jax.numpy as jnp
from jax import lax
from jax.experimental import pallas as pl
from jax.experimental.pallas import tpu as pltpu

N_DEV = 8

DX, DY, DZ = 1, 3, 4
ORDERS = ((DX, DY, DZ), (DY, DZ, DX), (DZ, DX, DY))
M_SPLITS = (352, 336, 336)
ROW_OFFS = (0, 352, 688)

STEPS = (
    (0, 0, 1),
    (1, 0, 1), (1, 1, 1),
    (2, 0, 1), (2, 1, 1), (2, 2, 1), (2, 3, 1),
)


def _cum_masks(order):
    m0, m1, m2 = order
    return (0, m0, m1, m1 ^ m0, m2, m2 ^ m0, m2 ^ m1, m2 ^ m1 ^ m0)


MASKS = tuple(_cum_masks(o) for o in ORDERS)


def kernel(A, B):
    m_per, k = A.shape
    k2, n = B.shape

    def body(a_ref, b_ref, dummy_ref, out_ref, b16_ref, g0, g1, g2,
             s0, s1, s2, send_sems, recv_sems, copy_sems):
        del dummy_ref
        my = lax.axis_index("i")
        gbufs = (g0, g1, g2)
        stages = (s0, s1, s2)

        barrier_sem = pltpu.get_barrier_semaphore()
        for d in (DX, DY, DZ):
            pl.semaphore_signal(
                barrier_sem, inc=1,
                device_id=(jnp.bitwise_xor(my, d),),
                device_id_type=pl.DeviceIdType.MESH,
            )
        pl.semaphore_wait(barrier_sem, 3)

        b16_ref[...] = b_ref[...].astype(jnp.bfloat16)
        for hc in range(3):
            gbufs[hc][0] = a_ref[
                pl.ds(ROW_OFFS[hc], M_SPLITS[hc]), :
            ].astype(jnp.bfloat16)

        def store(hc, j):
            if j >= 2:
                pltpu.make_async_copy(
                    stages[hc].at[j % 2],
                    out_ref.at[pl.ds(0, M_SPLITS[hc])],
                    copy_sems.at[hc, j - 2],
                ).wait()
            stages[hc][j % 2] = jnp.dot(
                gbufs[hc][j], b16_ref[...],
                preferred_element_type=jnp.float32,
            )
            origin = jnp.bitwise_xor(my, MASKS[hc][j])
            pltpu.make_async_copy(
                stages[hc].at[j % 2],
                out_ref.at[pl.ds(origin * m_per + ROW_OFFS[hc], M_SPLITS[hc])],
                copy_sems.at[hc, j],
            ).start()

        def start_step(hc, step):
            r, s, w = STEPS[step]
            partner = jnp.bitwise_xor(my, ORDERS[hc][r])
            rdma = pltpu.make_async_remote_copy(
                src_ref=gbufs[hc].at[pl.ds(s, w)],
                dst_ref=gbufs[hc].at[pl.ds(s + 2 ** r, w)],
                send_sem=send_sems.at[hc, step],
                recv_sem=recv_sems.at[hc, step],
                device_id=(partner,),
                device_id_type=pl.DeviceIdType.MESH,
            )
            rdma.start()
            return rdma

        rd0 = [start_step(hc, 0) for hc in range(3)]
        for hc in range(3):
            store(hc, 0)
        for hc in range(3):
            rd0[hc].wait()

        rd1 = [start_step(hc, 1) for hc in range(3)]
        rd2 = [start_step(hc, 2) for hc in range(3)]
        for hc in range(3):
            store(hc, 1)
        for hc in range(3):
            rd1[hc].wait()
            store(hc, 2)
        for hc in range(3):
            rd2[hc].wait()

        rds = [[start_step(hc, s) for hc in range(3)] for s in range(3, 7)]
        for hc in range(3):
            store(hc, 3)
        for i in range(4):
            for hc in range(3):
                rds[i][hc].wait()
                store(hc, 4 + i)

        for hc in range(3):
            for j in (6, 7):
                pltpu.make_async_copy(
                    stages[hc].at[j % 2],
                    out_ref.at[pl.ds(0, M_SPLITS[hc])],
                    copy_sems.at[hc, j],
                ).wait()

    return pl.pallas_call(
        body,
        out_shape=jax.ShapeDtypeStruct((N_DEV * m_per, n), jnp.float32),
        in_specs=[
            pl.BlockSpec(memory_space=pltpu.VMEM),
            pl.BlockSpec(memory_space=pltpu.VMEM),
            pl.BlockSpec(memory_space=pl.ANY),
        ],
        out_specs=pl.BlockSpec(memory_space=pl.ANY),
        input_output_aliases={2: 0},
        scratch_shapes=[
            pltpu.VMEM((k2, n), jnp.bfloat16),
            pltpu.VMEM((N_DEV, M_SPLITS[0], k), jnp.bfloat16),
            pltpu.VMEM((N_DEV, M_SPLITS[1], k), jnp.bfloat16),
            pltpu.VMEM((N_DEV, M_SPLITS[2], k), jnp.bfloat16),
            pltpu.VMEM((2, M_SPLITS[0], n), jnp.float32),
            pltpu.VMEM((2, M_SPLITS[1], n), jnp.float32),
            pltpu.VMEM((2, M_SPLITS[2], n), jnp.float32),
            pltpu.SemaphoreType.DMA((3, 7)),
            pltpu.SemaphoreType.DMA((3, 7)),
            pltpu.SemaphoreType.DMA((3, 8)),
        ],
        compiler_params=pltpu.CompilerParams(
            collective_id=0,
            vmem_limit_bytes=60 * 1024 * 1024,
        ),
    )(A, B, jnp.zeros((N_DEV * m_per, n), jnp.float32))


# device time: 61053 ns/iter; 1.2588x vs baseline; 1.2588x over previous
import jax
import jax.numpy as jnp
from jax import lax
from jax.experimental import pallas as pl
from jax.experimental.pallas import tpu as pltpu

N_DEV = 8

DX, DY, DZ = 1, 3, 4
ORDERS = ((DX, DY, DZ), (DY, DZ, DX), (DZ, DX, DY))
M_SPLITS = (352, 336, 336)
ROW_OFFS = (0, 352, 688)

STEPS = (
    (0, 0, 1),
    (1, 0, 1), (1, 1, 1),
    (2, 0, 1), (2, 1, 1), (2, 2, 1), (2, 3, 1),
)


def _cum_masks(order):
    m0, m1, m2 = order
    return (0, m0, m1, m1 ^ m0, m2, m2 ^ m0, m2 ^ m1, m2 ^ m1 ^ m0)


MASKS = tuple(_cum_masks(o) for o in ORDERS)


def kernel(A, B):
    m_per, k = A.shape
    k2, n = B.shape

    def body(a_ref, b_ref, out_ref, b16_ref, g0, g1, g2,
             s0, s1, s2, send_sems, recv_sems, copy_sems):
        my = lax.axis_index("i")
        gbufs = (g0, g1, g2)
        stages = (s0, s1, s2)

        barrier_sem = pltpu.get_barrier_semaphore()
        for d in (DX, DY, DZ):
            pl.semaphore_signal(
                barrier_sem, inc=1,
                device_id=(jnp.bitwise_xor(my, d),),
                device_id_type=pl.DeviceIdType.MESH,
            )
        pl.semaphore_wait(barrier_sem, 3)

        for hc in range(3):
            gbufs[hc][0] = a_ref[
                pl.ds(ROW_OFFS[hc], M_SPLITS[hc]), :
            ].astype(jnp.bfloat16)

        def store(hc, j):
            if j >= 2:
                pltpu.make_async_copy(
                    stages[hc].at[j % 2],
                    out_ref.at[pl.ds(0, M_SPLITS[hc])],
                    copy_sems.at[hc, j - 2],
                ).wait()
            stages[hc][j % 2] = jnp.dot(
                gbufs[hc][j], b16_ref[...],
                preferred_element_type=jnp.float32,
            )
            origin = jnp.bitwise_xor(my, MASKS[hc][j])
            pltpu.make_async_copy(
                stages[hc].at[j % 2],
                out_ref.at[pl.ds(origin * m_per + ROW_OFFS[hc], M_SPLITS[hc])],
                copy_sems.at[hc, j],
            ).start()

        def start_step(hc, step):
            r, s, w = STEPS[step]
            partner = jnp.bitwise_xor(my, ORDERS[hc][r])
            rdma = pltpu.make_async_remote_copy(
                src_ref=gbufs[hc].at[pl.ds(s, w)],
                dst_ref=gbufs[hc].at[pl.ds(s + 2 ** r, w)],
                send_sem=send_sems.at[hc, step],
                recv_sem=recv_sems.at[hc, step],
                device_id=(partner,),
                device_id_type=pl.DeviceIdType.MESH,
            )
            rdma.start()
            return rdma

        rdA = [start_step(hc, 0) for hc in range(3)]
        rdB = [start_step(hc, 1) for hc in range(3)]
        rdD = [start_step(hc, 3) for hc in range(3)]
        b16_ref[...] = b_ref[...].astype(jnp.bfloat16)
        for hc in range(3):
            store(hc, 0)
        rdC, rdE, rdF, rdG = [None] * 3, [None] * 3, [None] * 3, [None] * 3
        for hc in range(3):
            rdA[hc].wait()
            rdC[hc] = start_step(hc, 2)
            rdE[hc] = start_step(hc, 4)
        for hc in range(3):
            store(hc, 1)
        for hc in range(3):
            rdB[hc].wait()
            rdF[hc] = start_step(hc, 5)
        for hc in range(3):
            store(hc, 2)
        for hc in range(3):
            rdD[hc].wait()
            store(hc, 4)
        for hc in range(3):
            rdC[hc].wait()
            rdG[hc] = start_step(hc, 6)
        for hc in range(3):
            store(hc, 3)
        for rds, slot in ((rdE, 5), (rdF, 6), (rdG, 7)):
            for hc in range(3):
                rds[hc].wait()
                store(hc, slot)

        for hc in range(3):
            for j in (6, 7):
                pltpu.make_async_copy(
                    stages[hc].at[j % 2],
                    out_ref.at[pl.ds(0, M_SPLITS[hc])],
                    copy_sems.at[hc, j],
                ).wait()

    return pl.pallas_call(
        body,
        out_shape=jax.ShapeDtypeStruct((N_DEV * m_per, n), jnp.float32),
        in_specs=[
            pl.BlockSpec(memory_space=pltpu.VMEM),
            pl.BlockSpec(memory_space=pltpu.VMEM),
        ],
        out_specs=pl.BlockSpec(memory_space=pl.ANY),
        scratch_shapes=[
            pltpu.VMEM((k2, n), jnp.bfloat16),
            pltpu.VMEM((N_DEV, M_SPLITS[0], k), jnp.bfloat16),
            pltpu.VMEM((N_DEV, M_SPLITS[1], k), jnp.bfloat16),
            pltpu.VMEM((N_DEV, M_SPLITS[2], k), jnp.bfloat16),
            pltpu.VMEM((2, M_SPLITS[0], n), jnp.float32),
            pltpu.VMEM((2, M_SPLITS[1], n), jnp.float32),
            pltpu.VMEM((2, M_SPLITS[2], n), jnp.float32),
            pltpu.SemaphoreType.DMA((3, 7)),
            pltpu.SemaphoreType.DMA((3, 7)),
            pltpu.SemaphoreType.DMA((3, 8)),
        ],
        compiler_params=pltpu.CompilerParams(
            collective_id=0,
            vmem_limit_bytes=60 * 1024 * 1024,
        ),
    )(A, B)


# device time: 60240 ns/iter; 1.2758x vs baseline; 1.0135x over previous
import jax
import jax.numpy as jnp
from jax import lax
from jax.experimental import pallas as pl
from jax.experimental.pallas import tpu as pltpu

N_DEV = 8

DX, DY, DZ = 1, 3, 4
ORDERS = ((DX, DY, DZ), (DY, DZ, DX), (DZ, DX, DY))
M_SPLITS = (352, 336, 336)
ROW_OFFS = (0, 352, 688)

STEPS = (
    (0, 0, 1),
    (1, 0, 1), (1, 1, 1),
    (2, 0, 1), (2, 1, 1), (2, 2, 1), (2, 3, 1),
)


def _cum_masks(order):
    m0, m1, m2 = order
    return (0, m0, m1, m1 ^ m0, m2, m2 ^ m0, m2 ^ m1, m2 ^ m1 ^ m0)


MASKS = tuple(_cum_masks(o) for o in ORDERS)


def kernel(A, B):
    m_per, k = A.shape
    k2, n = B.shape

    def body(a_ref, b_ref, out_ref, b16_ref, g0, g1, g2,
             s0, s1, s2, send_sems, recv_sems, copy_sems):
        my = lax.axis_index("i")
        gbufs = (g0, g1, g2)
        stages = (s0, s1, s2)

        barrier_sem = pltpu.get_barrier_semaphore()
        for d in (DX, DY, DZ):
            pl.semaphore_signal(
                barrier_sem, inc=1,
                device_id=(jnp.bitwise_xor(my, d),),
                device_id_type=pl.DeviceIdType.MESH,
            )
        for hc in range(3):
            gbufs[hc][0] = a_ref[
                pl.ds(ROW_OFFS[hc], M_SPLITS[hc]), :
            ].astype(jnp.bfloat16)
        pl.semaphore_wait(barrier_sem, 3)

        def store(hc, j):
            if j >= 2:
                pltpu.make_async_copy(
                    stages[hc].at[j % 2],
                    out_ref.at[pl.ds(0, M_SPLITS[hc])],
                    copy_sems.at[hc, j - 2],
                ).wait()
            stages[hc][j % 2] = jnp.dot(
                gbufs[hc][j], b16_ref[...],
                preferred_element_type=jnp.float32,
            )
            origin = jnp.bitwise_xor(my, MASKS[hc][j])
            pltpu.make_async_copy(
                stages[hc].at[j % 2],
                out_ref.at[pl.ds(origin * m_per + ROW_OFFS[hc], M_SPLITS[hc])],
                copy_sems.at[hc, j],
            ).start()

        def start_step(hc, step):
            r, s, w = STEPS[step]
            partner = jnp.bitwise_xor(my, ORDERS[hc][r])
            rdma = pltpu.make_async_remote_copy(
                src_ref=gbufs[hc].at[pl.ds(s, w)],
                dst_ref=gbufs[hc].at[pl.ds(s + 2 ** r, w)],
                send_sem=send_sems.at[hc, step],
                recv_sem=recv_sems.at[hc, step],
                device_id=(partner,),
                device_id_type=pl.DeviceIdType.MESH,
            )
            rdma.start()
            return rdma

        rdA = [start_step(hc, 0) for hc in range(3)]
        rdB = [start_step(hc, 1) for hc in range(3)]
        rdD = [start_step(hc, 3) for hc in range(3)]
        b16_ref[...] = b_ref[...].astype(jnp.bfloat16)
        for hc in range(3):
            store(hc, 0)
        def start_g_half(hc, half):
            r0 = 176 * half
            nr = (M_SPLITS[hc] - 176) if half else 176
            partner = jnp.bitwise_xor(my, ORDERS[hc][2])
            rdma = pltpu.make_async_remote_copy(
                src_ref=gbufs[hc].at[3, pl.ds(r0, nr)],
                dst_ref=gbufs[hc].at[7, pl.ds(r0, nr)],
                send_sem=send_sems.at[hc, 6 + half],
                recv_sem=recv_sems.at[hc, 6 + half],
                device_id=(partner,),
                device_id_type=pl.DeviceIdType.MESH,
            )
            rdma.start()
            return rdma

        rdC, rdE, rdF = [None] * 3, [None] * 3, [None] * 3
        rdG1, rdG2 = [None] * 3, [None] * 3
        for hc in range(3):
            rdA[hc].wait()
            rdC[hc] = start_step(hc, 2)
            rdE[hc] = start_step(hc, 4)
        for hc in range(3):
            store(hc, 1)
        for hc in range(3):
            rdB[hc].wait()
            rdF[hc] = start_step(hc, 5)
        for hc in range(3):
            store(hc, 2)
        for hc in range(3):
            rdD[hc].wait()
            store(hc, 4)
        for hc in range(3):
            rdC[hc].wait()
            rdG1[hc] = start_g_half(hc, 0)
            rdG2[hc] = start_g_half(hc, 1)
        for hc in range(3):
            store(hc, 3)
        for rds, slot in ((rdE, 5), (rdF, 6)):
            for hc in range(3):
                rds[hc].wait()
                store(hc, slot)
        for hc in range(3):
            rdG1[hc].wait()
            pltpu.make_async_copy(
                stages[hc].at[1],
                out_ref.at[pl.ds(0, M_SPLITS[hc])],
                copy_sems.at[hc, 5],
            ).wait()
            stages[hc][1, pl.ds(0, 176)] = jnp.dot(
                gbufs[hc][7, pl.ds(0, 176)], b16_ref[...],
                preferred_element_type=jnp.float32,
            )
        for hc in range(3):
            nr = M_SPLITS[hc] - 176
            rdG2[hc].wait()
            stages[hc][1, pl.ds(176, nr)] = jnp.dot(
                gbufs[hc][7, pl.ds(176, nr)], b16_ref[...],
                preferred_element_type=jnp.float32,
            )
            origin = jnp.bitwise_xor(my, MASKS[hc][7])
            pltpu.make_async_copy(
                stages[hc].at[1],
                out_ref.at[pl.ds(origin * m_per + ROW_OFFS[hc], M_SPLITS[hc])],
                copy_sems.at[hc, 7],
            ).start()

        for hc in range(3):
            for j in (6, 7):
                pltpu.make_async_copy(
                    stages[hc].at[j % 2],
                    out_ref.at[pl.ds(0, M_SPLITS[hc])],
                    copy_sems.at[hc, j],
                ).wait()

    return pl.pallas_call(
        body,
        out_shape=jax.ShapeDtypeStruct((N_DEV * m_per, n), jnp.float32),
        in_specs=[
            pl.BlockSpec(memory_space=pltpu.VMEM),
            pl.BlockSpec(memory_space=pltpu.VMEM),
        ],
        out_specs=pl.BlockSpec(memory_space=pl.ANY),
        scratch_shapes=[
            pltpu.VMEM((k2, n), jnp.bfloat16),
            pltpu.VMEM((N_DEV, M_SPLITS[0], k), jnp.bfloat16),
            pltpu.VMEM((N_DEV, M_SPLITS[1], k), jnp.bfloat16),
            pltpu.VMEM((N_DEV, M_SPLITS[2], k), jnp.bfloat16),
            pltpu.VMEM((2, M_SPLITS[0], n), jnp.float32),
            pltpu.VMEM((2, M_SPLITS[1], n), jnp.float32),
            pltpu.VMEM((2, M_SPLITS[2], n), jnp.float32),
            pltpu.SemaphoreType.DMA((3, 8)),
            pltpu.SemaphoreType.DMA((3, 8)),
            pltpu.SemaphoreType.DMA((3, 8)),
        ],
        compiler_params=pltpu.CompilerParams(
            collective_id=0,
            vmem_limit_bytes=60 * 1024 * 1024,
        ),
    )(A, B)
